# baseline (speedup 1.0000x reference)
import functools

import jax
import jax.numpy as jnp
from jax import lax
from jax.experimental import pallas as pl
from jax.experimental.pallas import tpu as pltpu
from jax.experimental.pallas import tpu_sc as plsc

VOCAB = 100000
EMB = 64
BATCH = 1024

_INFO = plsc.get_sparse_core_info()
_NC = _INFO.num_cores
_NS = _INFO.num_subcores
_NW = _NC * _NS
_B_PER_W = BATCH // _NW


def _sc_gather_t(table_t, idx, nb):
  mesh = plsc.VectorSubcoreMesh(core_axis_name="c", subcore_axis_name="s")

  nslot = 8
  n_per_w = nb // _NW

  @functools.partial(
      pl.kernel,
      mesh=mesh,
      out_type=jax.ShapeDtypeStruct((nb, EMB), jnp.float32),
      scratch_types=(
          [pltpu.VMEM((n_per_w,), jnp.int32)]
          + [pltpu.VMEM((EMB, 128), jnp.float32)] * nslot
          + [pltpu.VMEM((n_per_w, EMB), jnp.float32)]
          + [pltpu.SemaphoreType.DMA] * nslot
      ),
      compiler_params=pltpu.CompilerParams(needs_layout_passes=False),
  )
  def gather_kernel(table_hbm, idx_hbm, out_hbm, idx_v, *rest):
    bufs = rest[:nslot]
    out_loc = rest[nslot]
    sems = rest[nslot + 1:]
    wid = lax.axis_index("s") * _NC + lax.axis_index("c")
    base = wid * n_per_w
    pltpu.sync_copy(idx_hbm.at[pl.ds(base, n_per_w)], idx_v)

    lane16 = lax.iota(jnp.int32, 16)
    idx_scalars = []
    for chunk in range(n_per_w // 16):
      vec = idx_v[pl.ds(chunk * 16, 16)]
      for k in range(16):
        idx_scalars.append(jnp.max(jnp.where(lane16 == k, vec, 0)))

    def slab_copy(i, slot):
      v = idx_scalars[i]
      off = pl.multiple_of((v // 128) * 128, 128)
      return pltpu.make_async_copy(
          table_hbm.at[:, pl.ds(off, 128)], bufs[slot], sems[slot])

    for s in range(nslot):
      slab_copy(s, s).start()
    for i in range(n_per_w):
      slot = i % nslot
      slab_copy(i, slot).wait()
      v = idx_scalars[i]
      col = v - (v // 128) * 128
      col_v = jnp.full((16,), col, jnp.int32)
      for p in range(EMB // 16):
        rows = lane16 + p * 16
        vals = plsc.load_gather(bufs[slot], [rows, col_v])
        out_loc[i, pl.ds(16 * p, 16)] = vals
      if i + nslot < n_per_w:
        slab_copy(i + nslot, slot).start()

    pltpu.sync_copy(out_loc, out_hbm.at[pl.ds(base, n_per_w), :])

  return gather_kernel(table_t, idx)


_VBLK = 6144


def _mm_body(wt_ref, b_ref, embt_ref, out_ref):
  w_aug = jnp.concatenate([wt_ref[...], b_ref[...]], axis=0)
  out_ref[...] = lax.dot_general(
      w_aug, embt_ref[...],
      (((0,), (0,)), ((), ())),
      preferred_element_type=jnp.float32,
  )


def _mm_body_alias(wt_ref, b_ref, embt_ref, prev_ref, out_ref):
  del prev_ref
  _mm_body(wt_ref, b_ref, embt_ref, out_ref)


_HB = BATCH // 2


def _tc_project_half(embt_aug, lin_w_t, lin_b2d, half, prev=None):
  grid = (pl.cdiv(VOCAB, _VBLK),)
  in_specs = [
      pl.BlockSpec((EMB, _VBLK), lambda j: (0, j)),
      pl.BlockSpec((1, _VBLK), lambda j: (0, j)),
      pl.BlockSpec((EMB + 1, _HB), lambda j: (0, 0)),
  ]
  args = [lin_w_t, lin_b2d, embt_aug]
  kwargs = {}
  body = _mm_body
  if prev is not None:
    in_specs.append(pl.BlockSpec(memory_space=pl.ANY))
    args.append(prev)
    kwargs["input_output_aliases"] = {3: 0}
    body = _mm_body_alias
  return pl.pallas_call(
      body,
      grid=grid,
      in_specs=in_specs,
      out_specs=pl.BlockSpec((_VBLK, _HB), lambda j, h=half: (j, h)),
      out_shape=jax.ShapeDtypeStruct((VOCAB, BATCH), jnp.float32),
      **kwargs,
  )(*args)


def _embt_aug_half(emb_h):
  return jnp.concatenate(
      [emb_h.T, jnp.ones((1, _HB), jnp.float32)], axis=0)


@jax.jit
def kernel(center_id, emb_table, lin_w, lin_b):
  idx = center_id.astype(jnp.int32)
  table_t = emb_table.T
  lin_w_t = lin_w.T
  lin_b2d = lin_b.reshape(1, VOCAB)
  emb1 = _sc_gather_t(table_t, idx[:_HB], _HB)
  emb2 = _sc_gather_t(table_t, idx[_HB:], _HB)
  o1 = _tc_project_half(_embt_aug_half(emb1), lin_w_t, lin_b2d, half=0)
  o = _tc_project_half(_embt_aug_half(emb2), lin_w_t, lin_b2d, half=1, prev=o1)
  return o.T


# speedup vs baseline: 1.0983x; 1.0983x over previous
import functools

import jax
import jax.numpy as jnp
from jax import lax
from jax.experimental import pallas as pl
from jax.experimental.pallas import tpu as pltpu
from jax.experimental.pallas import tpu_sc as plsc

VOCAB = 100000
EMB = 64
BATCH = 1024

_INFO = plsc.get_sparse_core_info()
_NC = _INFO.num_cores
_NS = _INFO.num_subcores
_NW = _NC * _NS
_B_PER_W = BATCH // _NW


def _sc_gather_t(table_t, idx, nb):
  mesh = plsc.VectorSubcoreMesh(core_axis_name="c", subcore_axis_name="s")

  nslot = 12
  n_per_w = nb // _NW

  @functools.partial(
      pl.kernel,
      mesh=mesh,
      out_type=jax.ShapeDtypeStruct((nb, EMB), jnp.float32),
      scratch_types=(
          [pltpu.VMEM((n_per_w,), jnp.int32)]
          + [pltpu.VMEM((EMB, 128), jnp.float32)] * nslot
          + [pltpu.VMEM((n_per_w, EMB), jnp.float32)]
          + [pltpu.SemaphoreType.DMA] * nslot
      ),
      compiler_params=pltpu.CompilerParams(needs_layout_passes=False),
  )
  def gather_kernel(table_hbm, idx_hbm, out_hbm, idx_v, *rest):
    bufs = rest[:nslot]
    out_loc = rest[nslot]
    sems = rest[nslot + 1:]
    wid = lax.axis_index("s") * _NC + lax.axis_index("c")
    base = wid * n_per_w
    pltpu.sync_copy(idx_hbm.at[pl.ds(base, n_per_w)], idx_v)

    lane16 = lax.iota(jnp.int32, 16)
    idx_scalars = []
    for chunk in range(n_per_w // 16):
      vec = idx_v[pl.ds(chunk * 16, 16)]
      for k in range(16):
        idx_scalars.append(jnp.max(jnp.where(lane16 == k, vec, 0)))

    def slab_copy(i, slot):
      v = idx_scalars[i]
      off = pl.multiple_of((v // 128) * 128, 128)
      return pltpu.make_async_copy(
          table_hbm.at[:, pl.ds(off, 128)], bufs[slot], sems[slot])

    for s in range(nslot):
      slab_copy(s, s).start()
    for i in range(n_per_w):
      slot = i % nslot
      slab_copy(i, slot).wait()
      v = idx_scalars[i]
      col = v - (v // 128) * 128
      col_v = jnp.full((16,), col, jnp.int32)
      for p in range(EMB // 16):
        rows = lane16 + p * 16
        vals = plsc.load_gather(bufs[slot], [rows, col_v])
        out_loc[i, pl.ds(16 * p, 16)] = vals
      if i + nslot < n_per_w:
        slab_copy(i + nslot, slot).start()

    pltpu.sync_copy(out_loc, out_hbm.at[pl.ds(base, n_per_w), :])

  return gather_kernel(table_t, idx)


_VBLK = 6144


def _mm_body(wt_ref, b_ref, embt_ref, out_ref):
  w_aug = jnp.concatenate([wt_ref[...], b_ref[...]], axis=0)
  out_ref[...] = lax.dot_general(
      w_aug, embt_ref[...],
      (((0,), (0,)), ((), ())),
      preferred_element_type=jnp.float32,
  )


def _tc_project(embt_aug, lin_w_t, lin_b2d):
  grid = (pl.cdiv(VOCAB, _VBLK),)
  return pl.pallas_call(
      _mm_body,
      grid=grid,
      in_specs=[
          pl.BlockSpec((EMB, _VBLK), lambda j: (0, j)),
          pl.BlockSpec((1, _VBLK), lambda j: (0, j)),
          pl.BlockSpec((EMB + 1, BATCH), lambda j: (0, 0)),
      ],
      out_specs=pl.BlockSpec((_VBLK, BATCH), lambda j: (j, 0)),
      out_shape=jax.ShapeDtypeStruct((VOCAB, BATCH), jnp.float32),
  )(lin_w_t, lin_b2d, embt_aug)


@jax.jit
def kernel(center_id, emb_table, lin_w, lin_b):
  emb = _sc_gather_t(emb_table.T, center_id.astype(jnp.int32), BATCH)
  embt_aug = jnp.concatenate(
      [emb.T, jnp.ones((1, BATCH), jnp.float32)], axis=0)
  out_t = _tc_project(embt_aug, lin_w.T, lin_b.reshape(1, VOCAB))
  return out_t.T


# speedup vs baseline: 1.1024x; 1.0038x over previous
import functools

import jax
import jax.numpy as jnp
from jax import lax
from jax.experimental import pallas as pl
from jax.experimental.pallas import tpu as pltpu
from jax.experimental.pallas import tpu_sc as plsc

VOCAB = 100000
EMB = 64
BATCH = 1024

_INFO = plsc.get_sparse_core_info()
_NC = _INFO.num_cores
_NS = _INFO.num_subcores
_NW = _NC * _NS
_B_PER_W = BATCH // _NW


def _sc_gather_t(table_t, idx, nb):
  mesh = plsc.VectorSubcoreMesh(core_axis_name="c", subcore_axis_name="s")

  nslot = 12
  n_per_w = nb // _NW

  @functools.partial(
      pl.kernel,
      mesh=mesh,
      out_type=jax.ShapeDtypeStruct((nb, EMB), jnp.float32),
      scratch_types=(
          [pltpu.VMEM((n_per_w,), jnp.int32)]
          + [pltpu.VMEM((EMB, 128), jnp.float32)] * nslot
          + [pltpu.VMEM((n_per_w, EMB), jnp.float32)]
          + [pltpu.SemaphoreType.DMA] * nslot
      ),
      compiler_params=pltpu.CompilerParams(needs_layout_passes=False),
  )
  def gather_kernel(table_hbm, idx_hbm, out_hbm, idx_v, *rest):
    bufs = rest[:nslot]
    out_loc = rest[nslot]
    sems = rest[nslot + 1:]
    wid = lax.axis_index("s") * _NC + lax.axis_index("c")
    base = wid * n_per_w
    pltpu.sync_copy(idx_hbm.at[pl.ds(base, n_per_w)], idx_v)

    lane16 = lax.iota(jnp.int32, 16)
    idx_scalars = []
    for chunk in range(n_per_w // 16):
      vec = idx_v[pl.ds(chunk * 16, 16)]
      for k in range(16):
        idx_scalars.append(jnp.max(jnp.where(lane16 == k, vec, 0)))

    def slab_copy(i, slot):
      v = idx_scalars[i]
      off = pl.multiple_of((v // 128) * 128, 128)
      return pltpu.make_async_copy(
          table_hbm.at[:, pl.ds(off, 128)], bufs[slot], sems[slot])

    for s in range(nslot):
      slab_copy(s, s).start()
    for i in range(n_per_w):
      slot = i % nslot
      slab_copy(i, slot).wait()
      v = idx_scalars[i]
      col = v - (v // 128) * 128
      col_v = jnp.full((16,), col, jnp.int32)
      for p in range(EMB // 16):
        rows = lane16 + p * 16
        vals = plsc.load_gather(bufs[slot], [rows, col_v])
        out_loc[i, pl.ds(16 * p, 16)] = vals
      if i + nslot < n_per_w:
        slab_copy(i + nslot, slot).start()

    pltpu.sync_copy(out_loc, out_hbm.at[pl.ds(base, n_per_w), :])

  return gather_kernel(table_t, idx)


_VBLK = 6144


def _mm_body(wt_ref, b_ref, embt_ref, out_ref):
  w_aug = jnp.concatenate([wt_ref[...], b_ref[...]], axis=0)
  out_ref[...] = lax.dot_general(
      w_aug, embt_ref[...],
      (((0,), (0,)), ((), ())),
      preferred_element_type=jnp.float32,
  )


def _tc_project(embt_aug, lin_w_t, lin_b2d):
  grid = (pl.cdiv(VOCAB, _VBLK),)
  return pl.pallas_call(
      _mm_body,
      grid=grid,
      in_specs=[
          pl.BlockSpec((EMB, _VBLK), lambda j: (0, j)),
          pl.BlockSpec((1, _VBLK), lambda j: (0, j)),
          pl.BlockSpec((EMB + 1, BATCH), lambda j: (0, 0)),
      ],
      out_specs=pl.BlockSpec((_VBLK, BATCH), lambda j: (j, 0)),
      out_shape=jax.ShapeDtypeStruct((VOCAB, BATCH), jnp.float32),
      compiler_params=pltpu.CompilerParams(fuse_transposed_lhs_in_matmul=True),
  )(lin_w_t, lin_b2d, embt_aug)


@jax.jit
def kernel(center_id, emb_table, lin_w, lin_b):
  emb = _sc_gather_t(emb_table.T, center_id.astype(jnp.int32), BATCH)
  embt_aug = jnp.concatenate(
      [emb.T, jnp.ones((1, BATCH), jnp.float32)], axis=0)
  out_t = _tc_project(embt_aug, lin_w.T, lin_b.reshape(1, VOCAB))
  return out_t.T


# speedup vs baseline: 1.1034x; 1.0009x over previous
import functools

import jax
import jax.numpy as jnp
from jax import lax
from jax.experimental import pallas as pl
from jax.experimental.pallas import tpu as pltpu
from jax.experimental.pallas import tpu_sc as plsc

VOCAB = 100000
EMB = 64
BATCH = 1024

_INFO = plsc.get_sparse_core_info()
_NC = _INFO.num_cores
_NS = _INFO.num_subcores
_NW = _NC * _NS
_B_PER_W = BATCH // _NW


def _sc_gather_t(table_t, idx, nb):
  mesh = plsc.VectorSubcoreMesh(core_axis_name="c", subcore_axis_name="s")

  nslot = 12
  n_per_w = nb // _NW

  @functools.partial(
      pl.kernel,
      mesh=mesh,
      out_type=jax.ShapeDtypeStruct((nb, EMB), jnp.float32),
      scratch_types=(
          [pltpu.VMEM((n_per_w,), jnp.int32)]
          + [pltpu.VMEM((EMB, 128), jnp.float32)] * nslot
          + [pltpu.VMEM((n_per_w, EMB), jnp.float32)]
          + [pltpu.SemaphoreType.DMA] * nslot
      ),
      compiler_params=pltpu.CompilerParams(needs_layout_passes=False),
  )
  def gather_kernel(table_hbm, idx_hbm, out_hbm, idx_v, *rest):
    bufs = rest[:nslot]
    out_loc = rest[nslot]
    sems = rest[nslot + 1:]
    wid = lax.axis_index("s") * _NC + lax.axis_index("c")
    base = wid * n_per_w
    pltpu.sync_copy(idx_hbm.at[pl.ds(base, n_per_w)], idx_v)

    lane16 = lax.iota(jnp.int32, 16)
    idx_scalars = []
    for chunk in range(n_per_w // 16):
      vec = idx_v[pl.ds(chunk * 16, 16)]
      for k in range(16):
        idx_scalars.append(jnp.max(jnp.where(lane16 == k, vec, 0)))

    def slab_copy(i, slot):
      v = idx_scalars[i]
      off = pl.multiple_of((v // 128) * 128, 128)
      return pltpu.make_async_copy(
          table_hbm.at[:, pl.ds(off, 128)], bufs[slot], sems[slot])

    for s in range(nslot):
      slab_copy(s, s).start()
    for i in range(n_per_w):
      slot = i % nslot
      slab_copy(i, slot).wait()
      v = idx_scalars[i]
      col = v - (v // 128) * 128
      col_v = jnp.full((16,), col, jnp.int32)
      for p in range(EMB // 16):
        rows = lane16 + p * 16
        vals = plsc.load_gather(bufs[slot], [rows, col_v])
        out_loc[i, pl.ds(16 * p, 16)] = vals
      if i + nslot < n_per_w:
        slab_copy(i + nslot, slot).start()

    pltpu.sync_copy(out_loc, out_hbm.at[pl.ds(base, n_per_w), :])

  return gather_kernel(table_t, idx)


_VBLK = 6144


def _mm_body(wt_ref, b_ref, embt_ref, out_ref):
  w_aug = jnp.concatenate(
      [wt_ref[...], b_ref[...].reshape(1, _VBLK)], axis=0)
  out_ref[...] = lax.dot_general(
      w_aug, embt_ref[...],
      (((0,), (0,)), ((), ())),
      preferred_element_type=jnp.float32,
  )


def _tc_project(embt_aug, lin_w_t, lin_b1d):
  grid = (pl.cdiv(VOCAB, _VBLK),)
  return pl.pallas_call(
      _mm_body,
      grid=grid,
      in_specs=[
          pl.BlockSpec((EMB, _VBLK), lambda j: (0, j)),
          pl.BlockSpec((_VBLK,), lambda j: (j,)),
          pl.BlockSpec((EMB + 1, BATCH), lambda j: (0, 0)),
      ],
      out_specs=pl.BlockSpec((_VBLK, BATCH), lambda j: (j, 0)),
      out_shape=jax.ShapeDtypeStruct((VOCAB, BATCH), jnp.float32),
  )(lin_w_t, lin_b1d, embt_aug)


@jax.jit
def kernel(center_id, emb_table, lin_w, lin_b):
  emb = _sc_gather_t(emb_table.T, center_id.astype(jnp.int32), BATCH)
  embt_aug = jnp.concatenate(
      [emb.T, jnp.ones((1, BATCH), jnp.float32)], axis=0)
  out_t = _tc_project(embt_aug, lin_w.T, lin_b)
  return out_t.T
